# baseline (speedup 1.0000x reference)
import functools

import jax
import jax.numpy as jnp
from jax import lax
from jax.experimental import pallas as pl
from jax.experimental.pallas import tpu as pltpu
from jax.experimental.pallas import tpu_sc as plsc

_NEG_SLOPE = 0.01
_L = 16


def _dots_body(nf_ref, watt_ref, wag_ref, t_ref, g_ref, a_ref):
    x = nf_ref[...]
    d = x.shape[1]
    w3 = jnp.concatenate(
        [watt_ref[:d], watt_ref[d:], wag_ref[...]], axis=1)
    yt = jax.lax.dot_general(w3, x, (((0,), (1,)), ((), ())),
                             preferred_element_type=jnp.float32)
    t_ref[...] = yt[0]
    g_ref[...] = yt[1]
    a_ref[...] = yt[2]


def _node_dots(nf, w_att, w_ag):
    n, d = nf.shape
    rows = 16384
    grid = pl.cdiv(n, rows)
    return pl.pallas_call(
        _dots_body,
        grid=(grid,),
        in_specs=[
            pl.BlockSpec((rows, d), lambda i: (i, 0)),
            pl.BlockSpec((2 * d, 1), lambda i: (0, 0)),
            pl.BlockSpec((d, 1), lambda i: (0, 0)),
        ],
        out_specs=[pl.BlockSpec((rows,), lambda i: (i,))] * 3,
        out_shape=[jax.ShapeDtypeStruct((n,), jnp.float32)] * 3,
    )(nf, w_att, w_ag)


def _rot_copy(wid, src_hbm, dst_v, total, nsplit, sem):
    cs = total // nsplit
    cps = []
    for i in range(nsplit):
        off = lax.rem(wid + i, nsplit) * cs
        cps.append(pltpu.async_copy(src_hbm.at[pl.ds(off, cs)],
                                    dst_v.at[pl.ds(off, cs)], sem))
    return cps


def _sc_compose(tni, edge_flat, n_ag, deg, n_task):
    nw = 32
    edges_per = n_ag * deg // nw
    mesh = plsc.VectorSubcoreMesh(core_axis_name="c", subcore_axis_name="s")

    @functools.partial(
        pl.kernel,
        out_type=jax.ShapeDtypeStruct((n_ag * deg,), jnp.int32),
        mesh=mesh,
        compiler_params=pltpu.CompilerParams(needs_layout_passes=False),
        scratch_types=[
            pltpu.VMEM((n_task,), jnp.int32),
            pltpu.VMEM((edges_per,), jnp.int32),
            pltpu.VMEM((edges_per,), jnp.int32),
            pltpu.SemaphoreType.DMA,
            pltpu.SemaphoreType.DMA,
        ],
    )
    def body(tni_hbm, edge_hbm, ci_hbm, tni_v, edge_v, ci_v, sem_n, sem_e):
        wid = lax.axis_index("s") * 2 + lax.axis_index("c")
        e0 = wid * edges_per
        cps_n = _rot_copy(wid, tni_hbm, tni_v, n_task, 5, sem_n)
        cp_e = pltpu.async_copy(edge_hbm.at[pl.ds(e0, edges_per)], edge_v,
                                sem_e)
        with jax.named_scope("sc_a_stage"):
            for cp in cps_n:
                cp.wait()
            cp_e.wait()

        def cb(i, _):
            for u in range(8):
                off = (i * 8 + u) * _L
                ev = edge_v[pl.ds(off, _L)]
                ci_v[pl.ds(off, _L)] = plsc.load_gather(tni_v, [ev])
            return 0

        with jax.named_scope("sc_a_compose"):
            lax.fori_loop(0, edges_per // (8 * _L), cb, 0)
        pltpu.sync_copy(ci_v, ci_hbm.at[pl.ds(e0, edges_per)])

    return body(tni, edge_flat)


def _sc_attention(t_full, g_full, a_full, ci_flat, agi, n_ag, deg, n_task):
    n = t_full.shape[0]
    nw = 32
    ag_per = n_ag // nw
    edges_per = ag_per * deg
    nvec_row = deg // _L
    chunk = 128
    mesh = plsc.VectorSubcoreMesh(core_axis_name="c", subcore_axis_name="s")

    @functools.partial(
        pl.kernel,
        out_type=[
            jax.ShapeDtypeStruct((n_ag * deg,), jnp.float32),
            jax.ShapeDtypeStruct((n_ag,), jnp.float32),
        ],
        mesh=mesh,
        compiler_params=pltpu.CompilerParams(needs_layout_passes=False),
        scratch_types=[
            pltpu.VMEM((n,), jnp.float32),
            pltpu.VMEM((edges_per,), jnp.int32),
            pltpu.VMEM((ag_per,), jnp.int32),
            pltpu.VMEM((ag_per,), jnp.float32),
            pltpu.VMEM((edges_per,), jnp.float32),
            pltpu.VMEM((n_ag,), jnp.int32),
            pltpu.VMEM((n_ag,), jnp.float32),
            pltpu.SemaphoreType.DMA,
            pltpu.SemaphoreType.DMA,
            pltpu.SemaphoreType.DMA,
            pltpu.SemaphoreType.DMA,
            pltpu.SemaphoreType.DMA,
        ],
    )
    def body(t_hbm, g_hbm, a_hbm, ci_hbm, agi_hbm, pol_hbm, agp_hbm,
             t_v, ci_v, agi_v, g_v, pol_v, agall_v, aval_v,
             sem_t, sem_e, sem_a, sem_g, sem_v):
        wid = lax.axis_index("s") * 2 + lax.axis_index("c")
        ag0 = wid * ag_per

        pltpu.async_copy(agi_hbm.at[pl.ds(ag0, ag_per)], agi_v, sem_a).wait()
        cp_g = pltpu.async_copy(g_hbm.at[agi_v], g_v, sem_g)
        cps_t = _rot_copy(wid, t_hbm, t_v, n, 5, sem_t)
        cp_e = pltpu.async_copy(ci_hbm.at[pl.ds(ag0 * deg, edges_per)],
                                ci_v, sem_e)

        @pl.when(wid == 0)
        def _():
            pltpu.sync_copy(agi_hbm, agall_v)

            def fire(i, _):
                for j in range(8):
                    off = (i * 8 + j) * chunk
                    pltpu.async_copy(
                        a_hbm.at[agall_v.at[pl.ds(off, chunk)]],
                        aval_v.at[pl.ds(off, chunk)], sem_v)
                return 0

            lax.fori_loop(0, n_ag // (8 * chunk), fire, 0)

        with jax.named_scope("sc_stage_wait"):
            for cp in cps_t:
                cp.wait()
            cp_e.wait()
            cp_g.wait()

        def row_body(r):
            gvec = plsc.load_gather(g_v, [jnp.full((_L,), r, jnp.int32)])
            base = r * deg
            svs = []
            for k in range(nvec_row):
                ti = ci_v[pl.ds(base + k * _L, _L)]
                tv = plsc.load_gather(t_v, [ti])
                x = tv + gvec
                svs.append(jnp.where(x >= 0.0, x, _NEG_SLOPE * x))
            mv = svs[0]
            for k in range(1, nvec_row):
                mv = jnp.maximum(mv, svs[k])
            m = jnp.max(mv)
            es = [jnp.exp(s - m) for s in svs]
            tot = es[0]
            for k in range(1, nvec_row):
                tot = tot + es[k]
            ssum = jnp.sum(tot)
            for k in range(nvec_row):
                pol_v[pl.ds(base + k * _L, _L)] = es[k] / ssum

        def row_block(rb, _):
            for u in range(4):
                row_body(rb * 4 + u)
            return 0

        with jax.named_scope("sc_rows"):
            lax.fori_loop(0, ag_per // 4, row_block, 0)
        cp_p = pltpu.async_copy(pol_v, pol_hbm.at[pl.ds(ag0 * deg, edges_per)],
                                sem_e)

        @pl.when(wid == 0)
        def _():
            pltpu.make_async_copy(a_hbm.at[pl.ds(0, n_ag)], aval_v,
                                  sem_v).wait()

            nv = n_ag // _L

            def pass1(i, c):
                x = aval_v[pl.ds(i * _L, _L)]
                x = jnp.where(x >= 0.0, x, _NEG_SLOPE * x)
                aval_v[pl.ds(i * _L, _L)] = x
                return jnp.maximum(c, x)

            mv = lax.fori_loop(0, nv, pass1,
                               jnp.full((_L,), -1e30, jnp.float32))
            m = jnp.max(mv)

            def pass2(i, c):
                e = jnp.exp(aval_v[pl.ds(i * _L, _L)] - m)
                aval_v[pl.ds(i * _L, _L)] = e
                return c + e

            sv = lax.fori_loop(0, nv, pass2, jnp.zeros((_L,), jnp.float32))
            ssum = jnp.sum(sv)

            def pass3(i, _):
                aval_v[pl.ds(i * _L, _L)] = aval_v[pl.ds(i * _L, _L)] / ssum
                return 0

            with jax.named_scope("sc_agsm"):
                lax.fori_loop(0, nv, pass3, 0)
                pltpu.sync_copy(aval_v, agp_hbm)

        cp_p.wait()

    return body(t_full, g_full, a_full, ci_flat, agi)


def kernel(nf, ag_node_indices, task_node_indices, task_finished,
           edge_task_idx, W_att, W_ag):
    n, d = nf.shape
    n_ag, deg = edge_task_idx.shape
    n_task = task_node_indices.shape[0]
    ci_flat = _sc_compose(task_node_indices, edge_task_idx.reshape(-1),
                          n_ag, deg, n_task)
    t_full, g_full, a_full = _node_dots(nf, W_att, W_ag)
    pol_flat, agp = _sc_attention(
        t_full, g_full, a_full, ci_flat, ag_node_indices, n_ag, deg, n_task)
    return pol_flat.reshape(n_ag, deg), agp


# speedup vs baseline: 1.0046x; 1.0046x over previous
import functools

import jax
import jax.numpy as jnp
from jax import lax
from jax.experimental import pallas as pl
from jax.experimental.pallas import tpu as pltpu
from jax.experimental.pallas import tpu_sc as plsc

_NEG_SLOPE = 0.01
_L = 16


def _dots_body(nf_ref, watt_ref, wag_ref, t_ref, g_ref, a_ref):
    x = nf_ref[...]
    d = x.shape[1]
    w3 = jnp.concatenate(
        [watt_ref[:d], watt_ref[d:], wag_ref[...]], axis=1)
    yt = jax.lax.dot_general(w3, x, (((0,), (1,)), ((), ())),
                             preferred_element_type=jnp.float32)
    t_ref[...] = yt[0]
    g_ref[...] = yt[1]
    a_ref[...] = yt[2]


def _node_dots(nf, w_att, w_ag):
    n, d = nf.shape
    rows = 8192
    grid = pl.cdiv(n, rows)
    return pl.pallas_call(
        _dots_body,
        grid=(grid,),
        in_specs=[
            pl.BlockSpec((rows, d), lambda i: (i, 0)),
            pl.BlockSpec((2 * d, 1), lambda i: (0, 0)),
            pl.BlockSpec((d, 1), lambda i: (0, 0)),
        ],
        out_specs=[pl.BlockSpec((rows,), lambda i: (i,))] * 3,
        out_shape=[jax.ShapeDtypeStruct((n,), jnp.float32)] * 3,
    )(nf, w_att, w_ag)


def _rot_copy(wid, src_hbm, dst_v, total, nsplit, sem):
    cs = total // nsplit
    cps = []
    for i in range(nsplit):
        off = lax.rem(wid + i, nsplit) * cs
        cps.append(pltpu.async_copy(src_hbm.at[pl.ds(off, cs)],
                                    dst_v.at[pl.ds(off, cs)], sem))
    return cps


def _sc_compose(tni, edge_flat, n_ag, deg, n_task):
    nw = 32
    edges_per = n_ag * deg // nw
    mesh = plsc.VectorSubcoreMesh(core_axis_name="c", subcore_axis_name="s")

    @functools.partial(
        pl.kernel,
        out_type=jax.ShapeDtypeStruct((n_ag * deg,), jnp.int32),
        mesh=mesh,
        compiler_params=pltpu.CompilerParams(needs_layout_passes=False),
        scratch_types=[
            pltpu.VMEM((n_task,), jnp.int32),
            pltpu.VMEM((edges_per,), jnp.int32),
            pltpu.VMEM((edges_per,), jnp.int32),
            pltpu.SemaphoreType.DMA,
            pltpu.SemaphoreType.DMA,
        ],
    )
    def body(tni_hbm, edge_hbm, ci_hbm, tni_v, edge_v, ci_v, sem_n, sem_e):
        wid = lax.axis_index("s") * 2 + lax.axis_index("c")
        e0 = wid * edges_per
        cps_n = _rot_copy(wid, tni_hbm, tni_v, n_task, 5, sem_n)
        cp_e = pltpu.async_copy(edge_hbm.at[pl.ds(e0, edges_per)], edge_v,
                                sem_e)
        with jax.named_scope("sc_a_stage"):
            for cp in cps_n:
                cp.wait()
            cp_e.wait()

        def cb(i, _):
            for u in range(8):
                off = (i * 8 + u) * _L
                ev = edge_v[pl.ds(off, _L)]
                ci_v[pl.ds(off, _L)] = plsc.load_gather(tni_v, [ev])
            return 0

        with jax.named_scope("sc_a_compose"):
            lax.fori_loop(0, edges_per // (8 * _L), cb, 0)
        pltpu.sync_copy(ci_v, ci_hbm.at[pl.ds(e0, edges_per)])

    return body(tni, edge_flat)


def _sc_attention(t_full, g_full, a_full, ci_flat, agi, n_ag, deg, n_task):
    n = t_full.shape[0]
    nw = 32
    ag_per = n_ag // nw
    edges_per = ag_per * deg
    nvec_row = deg // _L
    chunk = 128
    mesh = plsc.VectorSubcoreMesh(core_axis_name="c", subcore_axis_name="s")

    @functools.partial(
        pl.kernel,
        out_type=[
            jax.ShapeDtypeStruct((n_ag * deg,), jnp.float32),
            jax.ShapeDtypeStruct((n_ag,), jnp.float32),
        ],
        mesh=mesh,
        compiler_params=pltpu.CompilerParams(needs_layout_passes=False),
        scratch_types=[
            pltpu.VMEM((n,), jnp.float32),
            pltpu.VMEM((edges_per,), jnp.int32),
            pltpu.VMEM((ag_per,), jnp.int32),
            pltpu.VMEM((ag_per,), jnp.float32),
            pltpu.VMEM((edges_per,), jnp.float32),
            pltpu.VMEM((n_ag,), jnp.int32),
            pltpu.VMEM((n_ag,), jnp.float32),
            pltpu.SemaphoreType.DMA,
            pltpu.SemaphoreType.DMA,
            pltpu.SemaphoreType.DMA,
            pltpu.SemaphoreType.DMA,
            pltpu.SemaphoreType.DMA,
        ],
    )
    def body(t_hbm, g_hbm, a_hbm, ci_hbm, agi_hbm, pol_hbm, agp_hbm,
             t_v, ci_v, agi_v, g_v, pol_v, agall_v, aval_v,
             sem_t, sem_e, sem_a, sem_g, sem_v):
        wid = lax.axis_index("s") * 2 + lax.axis_index("c")
        ag0 = wid * ag_per

        pltpu.async_copy(agi_hbm.at[pl.ds(ag0, ag_per)], agi_v, sem_a).wait()
        cp_g = pltpu.async_copy(g_hbm.at[agi_v], g_v, sem_g)
        cps_t = _rot_copy(wid, t_hbm, t_v, n, 5, sem_t)
        cp_e = pltpu.async_copy(ci_hbm.at[pl.ds(ag0 * deg, edges_per)],
                                ci_v, sem_e)

        @pl.when(wid == 0)
        def _():
            pltpu.sync_copy(agi_hbm, agall_v)

            def fire(i, _):
                for j in range(8):
                    off = (i * 8 + j) * chunk
                    pltpu.async_copy(
                        a_hbm.at[agall_v.at[pl.ds(off, chunk)]],
                        aval_v.at[pl.ds(off, chunk)], sem_v)
                return 0

            lax.fori_loop(0, n_ag // (8 * chunk), fire, 0)

        with jax.named_scope("sc_stage_wait"):
            for cp in cps_t:
                cp.wait()
            cp_e.wait()
            cp_g.wait()

        def row_body(r):
            gvec = plsc.load_gather(g_v, [jnp.full((_L,), r, jnp.int32)])
            base = r * deg
            svs = []
            for k in range(nvec_row):
                ti = ci_v[pl.ds(base + k * _L, _L)]
                tv = plsc.load_gather(t_v, [ti])
                x = tv + gvec
                svs.append(jnp.where(x >= 0.0, x, _NEG_SLOPE * x))
            mv = svs[0]
            for k in range(1, nvec_row):
                mv = jnp.maximum(mv, svs[k])
            m = jnp.max(mv)
            es = [jnp.exp(s - m) for s in svs]
            tot = es[0]
            for k in range(1, nvec_row):
                tot = tot + es[k]
            ssum = jnp.sum(tot)
            for k in range(nvec_row):
                pol_v[pl.ds(base + k * _L, _L)] = es[k] / ssum

        def row_block(rb, _):
            for u in range(4):
                row_body(rb * 4 + u)
            return 0

        with jax.named_scope("sc_rows"):
            lax.fori_loop(0, ag_per // 4, row_block, 0)
        cp_p = pltpu.async_copy(pol_v, pol_hbm.at[pl.ds(ag0 * deg, edges_per)],
                                sem_e)

        @pl.when(wid == 0)
        def _():
            pltpu.make_async_copy(a_hbm.at[pl.ds(0, n_ag)], aval_v,
                                  sem_v).wait()

            nv = n_ag // _L

            def pass1(i, c):
                x = aval_v[pl.ds(i * _L, _L)]
                x = jnp.where(x >= 0.0, x, _NEG_SLOPE * x)
                aval_v[pl.ds(i * _L, _L)] = x
                return jnp.maximum(c, x)

            mv = lax.fori_loop(0, nv, pass1,
                               jnp.full((_L,), -1e30, jnp.float32))
            m = jnp.max(mv)

            def pass2(i, c):
                e = jnp.exp(aval_v[pl.ds(i * _L, _L)] - m)
                aval_v[pl.ds(i * _L, _L)] = e
                return c + e

            sv = lax.fori_loop(0, nv, pass2, jnp.zeros((_L,), jnp.float32))
            ssum = jnp.sum(sv)

            def pass3(i, _):
                aval_v[pl.ds(i * _L, _L)] = aval_v[pl.ds(i * _L, _L)] / ssum
                return 0

            with jax.named_scope("sc_agsm"):
                lax.fori_loop(0, nv, pass3, 0)
                pltpu.sync_copy(aval_v, agp_hbm)

        cp_p.wait()

    return body(t_full, g_full, a_full, ci_flat, agi)


def kernel(nf, ag_node_indices, task_node_indices, task_finished,
           edge_task_idx, W_att, W_ag):
    n, d = nf.shape
    n_ag, deg = edge_task_idx.shape
    n_task = task_node_indices.shape[0]
    ci_flat = _sc_compose(task_node_indices, edge_task_idx.reshape(-1),
                          n_ag, deg, n_task)
    t_full, g_full, a_full = _node_dots(nf, W_att, W_ag)
    pol_flat, agp = _sc_attention(
        t_full, g_full, a_full, ci_flat, ag_node_indices, n_ag, deg, n_task)
    return pol_flat.reshape(n_ag, deg), agp
